# baseline (speedup 1.0000x reference)
import functools

import jax
import jax.numpy as jnp
from jax import lax
from jax.experimental import pallas as pl
from jax.experimental.pallas import tpu as pltpu
from jax.experimental.pallas import tpu_sc as plsc

EMBED_DIM = 64
NUM_CORES = 2
NUM_SUBCORES = 16
NUM_WORKERS = NUM_CORES * NUM_SUBCORES
LANES = 16


@functools.lru_cache(maxsize=None)
def _make_kernel(seq: int, batch: int, vocab: int):
    bw = batch // NUM_WORKERS
    assert bw == 128 and seq % 2 == 0

    mesh = plsc.VectorSubcoreMesh(core_axis_name="c", subcore_axis_name="s")

    @functools.partial(
        pl.kernel,
        mesh=mesh,
        out_type=jax.ShapeDtypeStruct((seq, EMBED_DIM, batch), jnp.float32),
        scratch_types=[
            pltpu.VMEM((seq, bw), jnp.int32),
            pltpu.VMEM((seq, bw), jnp.int32),
            pltpu.VMEM((bw, 128), jnp.float32),
            pltpu.VMEM((bw, 128), jnp.float32),
            pltpu.VMEM((EMBED_DIM, bw), jnp.float32),
            pltpu.VMEM((EMBED_DIM, bw), jnp.float32),
            pltpu.SemaphoreType.DMA,
            pltpu.SemaphoreType.DMA,
            pltpu.SemaphoreType.DMA,
            pltpu.SemaphoreType.DMA,
        ],
        compiler_params=pltpu.CompilerParams(use_tc_tiling_on_sc=True, needs_layout_passes=False),
    )
    def gather_kernel(xt_hbm, table2_hbm, out_hbm, xv, pidx,
                      pair0, pair1, outv0, outv1, gsem0, gsem1, ssem0, ssem1):
        wid = lax.axis_index("s") * NUM_CORES + lax.axis_index("c")
        b0 = wid * bw
        pair = (pair0, pair1)
        outv = (outv0, outv1)
        gsem = (gsem0, gsem1)
        ssem = (ssem0, ssem1)

        pltpu.sync_copy(xt_hbm.at[:, pl.ds(b0, bw)], xv)

        def idx_body(s, carry):
            for c in range(bw // LANES):
                tok = xv[s, pl.ds(c * LANES, LANES)]
                pidx[s, pl.ds(c * LANES, LANES)] = tok >> 1
            return carry

        lax.fori_loop(0, seq, idx_body, 0)

        def g_start(s, p):
            pltpu.make_async_copy(
                table2_hbm.at[pidx.at[s]], pair[p], gsem[p]
            ).start()

        def g_wait(p):
            pltpu.make_async_copy(
                table2_hbm.at[pidx.at[0]], pair[p], gsem[p]
            ).wait()

        def s_start(s, p):
            pltpu.make_async_copy(
                outv[p], out_hbm.at[s, :, pl.ds(b0, bw)], ssem[p]
            ).start()

        def s_wait(p):
            pltpu.make_async_copy(
                outv[p], out_hbm.at[0, :, pl.ds(b0, bw)], ssem[p]
            ).wait()

        iota = lax.iota(jnp.int32, LANES)

        def transpose_select(s, p):
            svec = jnp.full((LANES,), 0, jnp.int32) + s

            def c_body(c, carry):
                rows = iota + c * LANES
                tok = plsc.load_gather(xv, [svec, rows])
                hcol = (tok & 1) * EMBED_DIM

                def d_body(dblk, carry2):
                    for j in range(8):
                        d = dblk * 8 + j
                        v = plsc.load_gather(pair[p], [rows, hcol + d])
                        outv[p][d, pl.ds(c * LANES, LANES)] = v
                    return carry2

                lax.fori_loop(0, EMBED_DIM // 8, d_body, 0)
                return carry

            lax.fori_loop(0, bw // LANES, c_body, 0)

        g_start(0, 0)
        g_start(1, 1)
        for p in range(2):
            g_wait(p)
            transpose_select(p, p)
            s_start(p, p)
            g_start(p + 2, p)

        def body(i, carry):
            for p in range(2):
                s = 2 * i + p
                g_wait(p)
                s_wait(p)
                transpose_select(s, p)
                s_start(s, p)
                g_start(s + 2, p)
            return carry

        lax.fori_loop(1, seq // 2 - 1, body, 0)

        for p in range(2):
            s = seq - 2 + p
            g_wait(p)
            s_wait(p)
            transpose_select(s, p)
            s_start(s, p)
        for p in range(2):
            s_wait(p)

    return gather_kernel


def kernel(x, table):
    batch, seq = x.shape
    vocab = table.shape[0]
    xt = x.T.astype(jnp.int32)
    table2 = table.reshape(vocab // 2, 2 * EMBED_DIM)
    out_t = _make_kernel(seq, batch, vocab)(xt, table2)
    return out_t.transpose(2, 0, 1)


# speedup vs baseline: 1.3370x; 1.3370x over previous
import functools

import jax
import jax.numpy as jnp
from jax import lax
from jax.experimental import pallas as pl
from jax.experimental.pallas import tpu as pltpu
from jax.experimental.pallas import tpu_sc as plsc

EMBED_DIM = 64
NUM_CORES = 2
NUM_SUBCORES = 16
NUM_WORKERS = NUM_CORES * NUM_SUBCORES
LANES = 16


@functools.lru_cache(maxsize=None)
def _make_kernel(seq: int, batch: int, vocab: int):
    bw = batch // NUM_WORKERS
    assert bw == 128 and seq % 2 == 0

    mesh = plsc.VectorSubcoreMesh(core_axis_name="c", subcore_axis_name="s")

    @functools.partial(
        pl.kernel,
        mesh=mesh,
        out_type=jax.ShapeDtypeStruct((seq, EMBED_DIM, batch), jnp.float32),
        scratch_types=[
            pltpu.VMEM((seq, bw), jnp.int32),
            pltpu.VMEM((seq, bw), jnp.int32),
            pltpu.VMEM((bw, 128), jnp.float32),
            pltpu.VMEM((bw, 128), jnp.float32),
            pltpu.VMEM((EMBED_DIM, bw), jnp.float32),
            pltpu.VMEM((EMBED_DIM, bw), jnp.float32),
            pltpu.SemaphoreType.DMA,
            pltpu.SemaphoreType.DMA,
            pltpu.SemaphoreType.DMA,
            pltpu.SemaphoreType.DMA,
        ],
        compiler_params=pltpu.CompilerParams(use_tc_tiling_on_sc=True, needs_layout_passes=False),
    )
    def gather_kernel(xt_hbm, table2_hbm, out_hbm, xv, pidx,
                      pair0, pair1, outv0, outv1, gsem0, gsem1, ssem0, ssem1):
        wid = lax.axis_index("s") * NUM_CORES + lax.axis_index("c")
        b0 = wid * bw
        pair = (pair0, pair1)
        outv = (outv0, outv1)
        gsem = (gsem0, gsem1)
        ssem = (ssem0, ssem1)

        pltpu.sync_copy(xt_hbm.at[:, pl.ds(b0, bw)], xv)

        def idx_body(s, carry):
            for c in range(bw // LANES):
                tok = xv[s, pl.ds(c * LANES, LANES)]
                pidx[s, pl.ds(c * LANES, LANES)] = tok >> 1
            return carry

        lax.fori_loop(0, seq, idx_body, 0)

        def g_start(s, p):
            pltpu.make_async_copy(
                table2_hbm.at[pidx.at[s]], pair[p], gsem[p]
            ).start()

        def g_wait(p):
            pltpu.make_async_copy(
                table2_hbm.at[pidx.at[0]], pair[p], gsem[p]
            ).wait()

        def s_start(s, p):
            pltpu.make_async_copy(
                outv[p], out_hbm.at[s, :, pl.ds(b0, bw)], ssem[p]
            ).start()

        def s_wait(p):
            pltpu.make_async_copy(
                outv[p], out_hbm.at[0, :, pl.ds(b0, bw)], ssem[p]
            ).wait()

        iota = lax.iota(jnp.int32, LANES)

        def transpose_select(s, p):
            @plsc.parallel_loop(0, bw // LANES, step=1)
            def body(c):
                rows = iota + c * LANES
                tok = xv[s, pl.ds(c * LANES, LANES)]
                hcol = (tok & 1) * EMBED_DIM
                for d in range(EMBED_DIM):
                    v = plsc.load_gather(pair[p], [rows, hcol + d])
                    outv[p][d, pl.ds(c * LANES, LANES)] = v

        g_start(0, 0)
        g_start(1, 1)
        for p in range(2):
            g_wait(p)
            transpose_select(p, p)
            s_start(p, p)
            g_start(p + 2, p)

        def body(i, carry):
            for p in range(2):
                s = 2 * i + p
                g_wait(p)
                s_wait(p)
                transpose_select(s, p)
                s_start(s, p)
                g_start(s + 2, p)
            return carry

        lax.fori_loop(1, seq // 2 - 1, body, 0)

        for p in range(2):
            s = seq - 2 + p
            g_wait(p)
            s_wait(p)
            transpose_select(s, p)
            s_start(s, p)
        for p in range(2):
            s_wait(p)

    return gather_kernel


def kernel(x, table):
    batch, seq = x.shape
    vocab = table.shape[0]
    xt = x.T.astype(jnp.int32)
    table2 = table.reshape(vocab // 2, 2 * EMBED_DIM)
    out_t = _make_kernel(seq, batch, vocab)(xt, table2)
    return out_t.transpose(2, 0, 1)
